# baseline (speedup 1.0000x reference)
import jax
import jax.numpy as jnp
from jax.experimental import pallas as pl

_TS = 256


def _add_kernel(x_ref, p_ref, o_ref):
    o_ref[...] = x_ref[...] + p_ref[...]


def kernel(x, pos_table):
    B, S, D = x.shape
    return pl.pallas_call(
        _add_kernel,
        grid=(S // _TS,),
        in_specs=[
            pl.BlockSpec((B, _TS, D), lambda i: (0, i, 0)),
            pl.BlockSpec((_TS, D), lambda i: (i, 0)),
        ],
        out_specs=pl.BlockSpec((B, _TS, D), lambda i: (0, i, 0)),
        out_shape=jax.ShapeDtypeStruct((B, S, D), x.dtype),
    )(x, pos_table[:S])


# speedup vs baseline: 1.0084x; 1.0084x over previous
import jax
import jax.numpy as jnp
from jax.experimental import pallas as pl

_TS = 512


def _add_kernel(x_ref, p_ref, o_ref):
    o_ref[...] = x_ref[...] + p_ref[...]


def kernel(x, pos_table):
    B, S, D = x.shape
    return pl.pallas_call(
        _add_kernel,
        grid=(S // _TS,),
        in_specs=[
            pl.BlockSpec((B, _TS, D), lambda i: (0, i, 0)),
            pl.BlockSpec((_TS, D), lambda i: (i, 0)),
        ],
        out_specs=pl.BlockSpec((B, _TS, D), lambda i: (0, i, 0)),
        out_shape=jax.ShapeDtypeStruct((B, S, D), x.dtype),
    )(x, pos_table[:S])
